# Initial kernel scaffold; baseline (speedup 1.0000x reference)
import jax
import jax.numpy as jnp
from jax.experimental import pallas as pl


def kernel(x):
    raise NotImplementedError("write your pallas kernel here")


# speedup vs baseline: 1.3491x; 1.3491x over previous
import jax
import jax.numpy as jnp
from jax.experimental import pallas as pl


_BR = 512


def _body(x_ref, o1_ref, o2_ref):
    v = x_ref[...]
    o1_ref[...] = v
    lead = jax.lax.broadcasted_iota(jnp.int32, v.shape, 0)
    o2_ref[...] = jnp.where(lead == 0, jnp.float32(2.0), v)


def kernel(x):
    n, r, c = x.shape
    grid = (r // _BR,)
    spec = pl.BlockSpec((n, _BR, c), lambda i: (0, i, 0))
    out1, out2 = pl.pallas_call(
        _body,
        grid=grid,
        in_specs=[spec],
        out_specs=[spec, spec],
        out_shape=[
            jax.ShapeDtypeStruct(x.shape, x.dtype),
            jax.ShapeDtypeStruct(x.shape, x.dtype),
        ],
    )(x)
    return (out1, out2)
